# baseline (speedup 1.0000x reference)
import jax
import jax.numpy as jnp
from jax.experimental import pallas as pl

N = 10000
E = 320000
IN_DIM = 128
HEADS = 8
OUT_CH = 16
ALPHA = 0.5


def _prep_body(x_ref, w_ref, a_ref, h_ref, acomb_ref):
    h = jnp.dot(x_ref[...], w_ref[0], preferred_element_type=jnp.float32)
    h_ref[...] = h[None]
    acomb_ref[...] = jnp.dot(h, a_ref[0], preferred_element_type=jnp.float32)[None]


def _combine_body(o1_ref, o2_ref, b_ref, out_ref):
    out_ref[...] = (1.0 - ALPHA) * o1_ref[...] + ALPHA * o2_ref[...] + b_ref[...]


def _block_att(att_src, att_dst):
    eye = jnp.eye(HEADS, dtype=att_src.dtype)[:, None, :]
    a_s = (att_src[:, :, None] * eye).reshape(IN_DIM, HEADS)
    a_d = (att_dst[:, :, None] * eye).reshape(IN_DIM, HEADS)
    return jnp.concatenate([a_s, a_d], axis=1)


def _gat_dir(h, acomb, g, t):
    e = acomb[g, :HEADS] + acomb[t, HEADS:]
    e = jnp.maximum(e, 0.2 * e)
    ex = jnp.exp(e)
    s = jax.ops.segment_sum(ex, t, num_segments=N)
    alpha = ex * (1.0 / (s + 1e-16))[t]
    msg = h[g].reshape(E, HEADS, OUT_CH) * alpha[:, :, None]
    out = jax.ops.segment_sum(msg.reshape(E, HEADS * OUT_CH), t, num_segments=N)
    return out


def kernel(x, edge_index, W1, att_src1, att_dst1, bias1,
           W2, att_src2, att_dst2, bias2):
    Wst = jnp.stack([W1, W2])
    Ast = jnp.stack([_block_att(att_src1, att_dst1),
                     _block_att(att_src2, att_dst2)])

    hs, acombs = pl.pallas_call(
        _prep_body,
        grid=(2,),
        in_specs=[
            pl.BlockSpec((N, IN_DIM), lambda d: (0, 0)),
            pl.BlockSpec((1, IN_DIM, IN_DIM), lambda d: (d, 0, 0)),
            pl.BlockSpec((1, IN_DIM, 16), lambda d: (d, 0, 0)),
        ],
        out_specs=[
            pl.BlockSpec((1, N, IN_DIM), lambda d: (d, 0, 0)),
            pl.BlockSpec((1, N, 16), lambda d: (d, 0, 0)),
        ],
        out_shape=[
            jax.ShapeDtypeStruct((2, N, IN_DIM), jnp.float32),
            jax.ShapeDtypeStruct((2, N, 16), jnp.float32),
        ],
    )(x, Wst, Ast)

    src = edge_index[0]
    dst = edge_index[1]
    out1 = _gat_dir(hs[0], acombs[0], src, dst)
    out2 = _gat_dir(hs[1], acombs[1], dst, src)

    bcomb = ((1.0 - ALPHA) * bias1 + ALPHA * bias2).reshape(1, IN_DIM)
    return pl.pallas_call(
        _combine_body,
        out_shape=jax.ShapeDtypeStruct((N, IN_DIM), jnp.float32),
    )(out1, out2, bcomb)
